# baseline (speedup 1.0000x reference)
import jax
import jax.numpy as jnp
from jax.experimental import pallas as pl

_K = 9


def _edge_kernel(x_ref, segr_ref, segc_ref, w_ref, idx_ref, attr_ref):
    A = 15
    L = 128
    E = w_ref.shape[1]
    INF = jnp.float32(jnp.inf)
    IMAX = jnp.int32(2147483647)

    x = x_ref[0]
    x0 = x[:, 0:1]
    x1 = x[:, 1:2]
    x2 = x[:, 2:3]
    sq_all = (x0 * x0 + x1 * x1) + x2 * x2
    sqc = jnp.transpose(sq_all)
    xm2 = -2.0 * x

    Q = None
    for a1 in range(A):
        r0 = a1 * L
        sqr = sq_all[r0:r0 + L, :]
        g2s = jax.lax.dot_general(xm2[r0:r0 + L, :], x[r0:, :],
                                  (((1,), (1,)), ((), ())),
                                  preferred_element_type=jnp.float32)
        for a2 in range(a1, A):
            c0 = a2 * L
            d2t = (sqr + sqc[:, c0:c0 + L]) + g2s[:, c0 - r0:c0 - r0 + L]
            Q = d2t if Q is None else jnp.minimum(Q, d2t)
    acc = jnp.minimum(Q, Q.T)

    d = jnp.sqrt(jnp.maximum(acc, 0.0))
    rowb = jax.lax.broadcasted_iota(jnp.int32, (L, L), 0)
    colb = jax.lax.broadcasted_iota(jnp.int32, (L, L), 1)
    d = jnp.where(rowb == colb, INF, d)

    sr = segr_ref[0]
    sc = segc_ref[0]
    sr = jnp.where(sr == 2, 1, sr)
    sc = jnp.where(sc == 2, 1, sc)
    same = sr == sc
    d_intra = jnp.where(same, d, INF)
    d_inter = jnp.where(same, INF, d)

    boff = pl.program_id(0) * L
    col32 = jax.lax.broadcasted_iota(jnp.int32, (L, 32), 1)

    def topk_accum(dm, res, base):
        key = jax.lax.bitcast_convert_type(dm, jnp.int32)
        for k in range(_K):
            vmin = jnp.min(key, axis=0, keepdims=True)
            idx = jnp.min(jnp.where(key == vmin, rowb, IMAX),
                          axis=0, keepdims=True)
            key = jnp.where(rowb == idx, IMAX, key)
            res = jnp.where(col32 == base + k,
                            jnp.transpose(idx) + boff, res)
        return res

    res = jnp.zeros((L, 32), jnp.int32)
    res = topk_accum(d_intra, res, 0)
    res = topk_accum(d_inter, res, 16)
    idx_ref[0] = res

    attr_ref[0, 0] = jnp.broadcast_to(w_ref[0:1, :], (L * _K, E))
    attr_ref[1, 0] = jnp.broadcast_to(w_ref[1:2, :], (L * _K, E))


def kernel(pos_heavyatom, aa, atom_types, mask_atoms, block_lengths,
           lengths, fragment_type, W_edge):
    B, L, A, _ = pos_heavyatom.shape
    E = W_edge.shape[1]
    K = _K

    Xp = jnp.transpose(pos_heavyatom, (0, 2, 1, 3)).reshape(B, A * L, 3)
    segr = fragment_type.reshape(B, L, 1)
    segc = fragment_type.reshape(B, 1, L)

    idx_out, attr4 = pl.pallas_call(
        _edge_kernel,
        grid=(B,),
        in_specs=[
            pl.BlockSpec((1, A * L, 3), lambda b: (b, 0, 0)),
            pl.BlockSpec((1, L, 1), lambda b: (b, 0, 0)),
            pl.BlockSpec((1, 1, L), lambda b: (b, 0, 0)),
            pl.BlockSpec((2, E), lambda b: (0, 0)),
        ],
        out_specs=[
            pl.BlockSpec((1, L, 32), lambda b: (b, 0, 0)),
            pl.BlockSpec((2, 1, L * K, E), lambda b: (0, b, 0, 0)),
        ],
        out_shape=[
            jax.ShapeDtypeStruct((B, L, 32), jnp.int32),
            jax.ShapeDtypeStruct((2, B, L * K, E), jnp.float32),
        ],
    )(Xp, segr, segc, W_edge)

    srcs = jnp.concatenate([idx_out[:, :, :K],
                            idx_out[:, :, 16:16 + K]], axis=0).reshape(-1)
    off = (jnp.arange(B, dtype=jnp.int32) * L)[:, None, None]
    dst = jnp.broadcast_to(jnp.arange(L, dtype=jnp.int32)[None, :, None] + off,
                           (B, L, K)).reshape(-1)
    edges = jnp.stack([srcs, jnp.concatenate([dst, dst])], axis=0)
    edge_attr = attr4.reshape(2 * B * L * K, E)

    block_id = jnp.repeat(jnp.arange(B * L, dtype=jnp.int32), A)
    batch_id = jnp.repeat(jnp.arange(B, dtype=jnp.int32), L)
    return (block_id, batch_id, edges, edge_attr)


# speedup vs baseline: 1.1866x; 1.1866x over previous
import jax
import jax.numpy as jnp
from jax.experimental import pallas as pl

_K = 9


def _edge_kernel(x_ref, segr_ref, segc_ref, w_ref, idx_ref, attr_ref):
    A = 15
    L = 128
    E = w_ref.shape[1]
    INF = jnp.float32(jnp.inf)
    IMAX = jnp.int32(2147483647)

    x = x_ref[0]
    x0 = x[:, 0:1]
    x1 = x[:, 1:2]
    x2 = x[:, 2:3]
    sq_all = (x0 * x0 + x1 * x1) + x2 * x2
    sqc = jnp.transpose(sq_all)
    xm2 = -2.0 * x

    Q = None
    for a1 in range(A):
        r0 = a1 * L
        sqr = sq_all[r0:r0 + L, :]
        g2s = jax.lax.dot_general(xm2[r0:r0 + L, :], x[r0:, :],
                                  (((1,), (1,)), ((), ())),
                                  preferred_element_type=jnp.float32)
        for a2 in range(a1, A):
            c0 = a2 * L
            d2t = (sqr + sqc[:, c0:c0 + L]) + g2s[:, c0 - r0:c0 - r0 + L]
            Q = d2t if Q is None else jnp.minimum(Q, d2t)
    acc = jnp.minimum(Q, Q.T)

    d = jnp.sqrt(jnp.maximum(acc, 0.0))
    rowb = jax.lax.broadcasted_iota(jnp.int32, (L, L), 0)
    colb = jax.lax.broadcasted_iota(jnp.int32, (L, L), 1)
    d = jnp.where(rowb == colb, INF, d)

    sr = segr_ref[0]
    sc = segc_ref[0]
    sr = jnp.where(sr == 2, 1, sr)
    sc = jnp.where(sc == 2, 1, sc)
    same = sr == sc
    d_intra = jnp.where(same, d, INF)
    d_inter = jnp.where(same, INF, d)

    boff = pl.program_id(0) * L
    row32 = jax.lax.broadcasted_iota(jnp.int32, (32, L), 0)

    def topk_accum(dm, res, base):
        key = jax.lax.bitcast_convert_type(dm, jnp.int32)
        for k in range(_K):
            vmin = jnp.min(key, axis=0, keepdims=True)
            idx = jnp.min(jnp.where(key == vmin, rowb, IMAX),
                          axis=0, keepdims=True)
            key = jnp.where(rowb == idx, IMAX, key)
            res = jnp.where(row32 == base + k,
                            jnp.broadcast_to(idx + boff, (32, L)), res)
        return res

    res = jnp.zeros((32, L), jnp.int32)
    res = topk_accum(d_intra, res, 0)
    res = topk_accum(d_inter, res, 16)
    idx_ref[0] = res

    attr_ref[0, 0] = jnp.broadcast_to(w_ref[0:1, :], (L * _K, E))
    attr_ref[1, 0] = jnp.broadcast_to(w_ref[1:2, :], (L * _K, E))


def kernel(pos_heavyatom, aa, atom_types, mask_atoms, block_lengths,
           lengths, fragment_type, W_edge):
    B, L, A, _ = pos_heavyatom.shape
    E = W_edge.shape[1]
    K = _K

    Xp = jnp.transpose(pos_heavyatom, (0, 2, 1, 3)).reshape(B, A * L, 3)
    segr = fragment_type.reshape(B, L, 1)
    segc = fragment_type.reshape(B, 1, L)

    idx_out, attr4 = pl.pallas_call(
        _edge_kernel,
        grid=(B,),
        in_specs=[
            pl.BlockSpec((1, A * L, 3), lambda b: (b, 0, 0)),
            pl.BlockSpec((1, L, 1), lambda b: (b, 0, 0)),
            pl.BlockSpec((1, 1, L), lambda b: (b, 0, 0)),
            pl.BlockSpec((2, E), lambda b: (0, 0)),
        ],
        out_specs=[
            pl.BlockSpec((1, 32, L), lambda b: (b, 0, 0)),
            pl.BlockSpec((2, 1, L * K, E), lambda b: (0, b, 0, 0)),
        ],
        out_shape=[
            jax.ShapeDtypeStruct((B, 32, L), jnp.int32),
            jax.ShapeDtypeStruct((2, B, L * K, E), jnp.float32),
        ],
    )(Xp, segr, segc, W_edge)

    srcs = jnp.concatenate([idx_out[:, :K, :],
                            idx_out[:, 16:16 + K, :]],
                           axis=0).transpose(0, 2, 1).reshape(-1)
    off = (jnp.arange(B, dtype=jnp.int32) * L)[:, None, None]
    dst = jnp.broadcast_to(jnp.arange(L, dtype=jnp.int32)[None, :, None] + off,
                           (B, L, K)).reshape(-1)
    edges = jnp.stack([srcs, jnp.concatenate([dst, dst])], axis=0)
    edge_attr = attr4.reshape(2 * B * L * K, E)

    block_id = jnp.repeat(jnp.arange(B * L, dtype=jnp.int32), A)
    batch_id = jnp.repeat(jnp.arange(B, dtype=jnp.int32), L)
    return (block_id, batch_id, edges, edge_attr)
